# Initial kernel scaffold; baseline (speedup 1.0000x reference)
import jax
import jax.numpy as jnp
from jax.experimental import pallas as pl


def kernel(l_data, m_data, h_data, targets, input_wh):
    raise NotImplementedError("write your pallas kernel here")


# speedup vs baseline: 10.4786x; 10.4786x over previous
import functools

import jax
import jax.numpy as jnp
import numpy as np
from jax.experimental import pallas as pl
from jax.experimental.pallas import tpu as pltpu

_ANCH = np.array(
    [[10, 13], [16, 30], [33, 23], [30, 61], [62, 45], [59, 119],
     [116, 90], [156, 198], [373, 326]], dtype=np.float32)
_GRIDW = (52, 26, 13)
_OFFS = (0, 8112, 10140)
_B = 8
_NT = 20


def _match_body(misc_ref, tgt_ref, nt_ref, valid_ref):
    iw = misc_ref[0]
    t = tgt_ref[...]
    x1 = t[:, :, 0:1]
    y1 = t[:, :, 1:2]
    x2 = t[:, :, 2:3]
    y2 = t[:, :, 3:4]
    w_n = x2 - x1
    h_n = y2 - y1
    valid = (w_n > 0) & (h_n > 0)
    cx = (x1 + x2) * 0.5
    cy = (y1 + y2) * 0.5
    w_px = w_n * iw
    h_px = h_n * iw
    ai = jax.lax.broadcasted_iota(jnp.int32, (_B, _NT, 9), 2)
    aw = _sel9(ai, _ANCH[:, 0])
    ah = _sel9(ai, _ANCH[:, 1])
    inter = jnp.minimum(w_px, aw) * jnp.minimum(h_px, ah)
    aiou = inter / (w_px * h_px + aw * ah - inter + 1e-9)
    m = jnp.max(aiou, axis=2, keepdims=True)
    astar = jnp.min(jnp.where(aiou == m, ai, 99), axis=2, keepdims=True)
    astar = jnp.clip(astar, 0, 8)
    s = astar // 3
    aloc = astar % 3
    gw = jnp.where(s == 0, _GRIDW[0], jnp.where(s == 1, _GRIDW[1], _GRIDW[2]))
    off = jnp.where(s == 0, _OFFS[0], jnp.where(s == 1, _OFFS[1], _OFFS[2]))
    gwf = gw.astype(jnp.float32)
    gi = jnp.clip((cx * gwf).astype(jnp.int32), 0, gw - 1)
    gj = jnp.clip((cy * gwf).astype(jnp.int32), 0, gw - 1)
    nt_ref[...] = off + (gj * gw + gi) * 3 + aloc
    valid_ref[...] = valid.astype(jnp.float32)


def _dense_body(misc_ref, data_ref, tgt_ref, nt_ref, val_ref, back_ref,
                comp_ref, *, W, H, OFF, A0):
    iw = misc_ref[0]
    stride = misc_ref[1]
    HW = W * H
    tgt = tgt_ref[0]
    gx1 = tgt[:, 0:1] * iw
    gy1 = tgt[:, 1:2] * iw
    gx2 = tgt[:, 2:3] * iw
    gy2 = tgt[:, 3:4] * iw
    area_g = (gx2 - gx1) * (gy2 - gy1)
    nt = nt_ref[0]
    vld = val_ref[0] > 0
    pos = jax.lax.broadcasted_iota(jnp.int32, (1, HW), 1)
    gxf = (pos % W).astype(jnp.float32)
    gyf = (pos // W).astype(jnp.float32)
    total = jnp.float32(0.0)
    comp = jnp.zeros((_NT, 85), jnp.float32)
    for a in range(3):
        txs = jax.nn.sigmoid(data_ref[0, a, 0:1, :])
        tys = jax.nn.sigmoid(data_ref[0, a, 1:2, :])
        tw = data_ref[0, a, 2:3, :]
        th = data_ref[0, a, 3:4, :]
        conf_logit = data_ref[0, a, 4:5, :]
        cx = (txs + gxf) * stride
        cy = (tys + gyf) * stride
        bw = A0[a][0] * jnp.exp(jnp.clip(tw, -10.0, 10.0))
        bh = A0[a][1] * jnp.exp(jnp.clip(th, -10.0, 10.0))
        bx1 = cx - bw * 0.5
        by1 = cy - bh * 0.5
        bx2 = cx + bw * 0.5
        by2 = cy + bh * 0.5
        ix1 = jnp.maximum(bx1, gx1)
        iy1 = jnp.maximum(by1, gy1)
        ix2 = jnp.minimum(bx2, gx2)
        iy2 = jnp.minimum(by2, gy2)
        inter = jnp.maximum(ix2 - ix1, 0.0) * jnp.maximum(iy2 - iy1, 0.0)
        area_b = (bx2 - bx1) * (by2 - by1)
        iou = inter / (area_b + area_g - inter + 1e-9)
        max_iou = jnp.max(iou, axis=0, keepdims=True)
        back0 = max_iou <= 0.5
        n_glob = OFF + pos * 3 + a
        eq = nt == n_glob
        fore = jnp.any(eq & vld, axis=0, keepdims=True)
        conf = jnp.clip(jax.nn.sigmoid(conf_logit), 1e-7, 1.0 - 1e-7)
        term = jnp.where(back0 & jnp.logical_not(fore),
                         -jnp.log(1.0 - conf), 0.0)
        total = total + jnp.sum(term)
        comp = comp + jax.lax.dot_general(
            eq.astype(jnp.float32), data_ref[0, a],
            (((1,), (1,)), ((), ())), preferred_element_type=jnp.float32)
    back_ref[0, 0, 0] = total
    comp_ref[0] = comp


def _sel9(idx, vals):
    out = jnp.full(idx.shape, vals[8], dtype=jnp.float32)
    for k in range(7, -1, -1):
        out = jnp.where(idx == k, jnp.float32(vals[k]), out)
    return out


def _loss_body(misc_ref, tgt_ref, nt_ref, val_ref, cl_ref, cm_ref, ch_ref,
               bl_ref, bm_ref, bh_ref, out_ref):
    iw = misc_ref[0]
    t = tgt_ref[...]
    x1 = t[:, :, 0:1]
    y1 = t[:, :, 1:2]
    x2 = t[:, :, 2:3]
    y2 = t[:, :, 3:4]
    cls = t[:, :, 4:5]
    n = nt_ref[...]
    vld = val_ref[...] > 0
    winner = jnp.full((_B, _NT, 1), -1, jnp.int32)
    for tp in range(_NT):
        n_tp = n[:, tp:tp + 1, :]
        v_tp = vld[:, tp:tp + 1, :]
        winner = jnp.where(v_tp & (n == n_tp), tp, winner)
    t_iota = jax.lax.broadcasted_iota(jnp.int32, (_B, _NT, 1), 1)
    active = vld & (winner == t_iota)
    actf = active.astype(jnp.float32)
    s = (n >= _OFFS[1]).astype(jnp.int32) + (n >= _OFFS[2]).astype(jnp.int32)
    off = jnp.where(s == 0, _OFFS[0], jnp.where(s == 1, _OFFS[1], _OFFS[2]))
    gw = jnp.where(s == 0, _GRIDW[0], jnp.where(s == 1, _GRIDW[1], _GRIDW[2]))
    r = n - off
    aloc = r % 3
    p = r // 3
    gi = p % gw
    gj = p // gw
    astar = s * 3 + aloc
    aw = _sel9(astar, _ANCH[:, 0])
    ah = _sel9(astar, _ANCH[:, 1])
    cxn = (x1 + x2) * 0.5
    cyn = (y1 + y2) * 0.5
    w_n = x2 - x1
    h_n = y2 - y1
    w_px = w_n * iw
    h_px = h_n * iw
    gwf = gw.astype(jnp.float32)
    tx = cxn * gwf - gi.astype(jnp.float32)
    ty = cyn * gwf - gj.astype(jnp.float32)
    twt = jnp.log(jnp.maximum(w_px, 1.0) / aw)
    tht = jnp.log(jnp.maximum(h_px, 1.0) / ah)
    scale = 2.0 - w_n * h_n
    comp = cl_ref[...] + cm_ref[...] + ch_ref[...]
    sig0 = jax.nn.sigmoid(comp)
    px = sig0[:, :, 0:1]
    py = sig0[:, :, 1:2]
    pw = comp[:, :, 2:3]
    ph = comp[:, :, 3:4]
    pc = sig0[:, :, 4:5]
    sf = scale * actf
    xy_loss = jnp.sum(sf * ((px - tx) ** 2 + (py - ty) ** 2)) * 0.5
    wh_loss = jnp.sum(sf * ((pw - twt) ** 2 + (ph - tht) ** 2)) * 0.5
    pcc = jnp.clip(pc, 1e-7, 1.0 - 1e-7)
    conf_fore = jnp.sum(actf * (-jnp.log(pcc)))
    c_iota = jax.lax.broadcasted_iota(jnp.int32, (_B, _NT, 85), 2)
    clsp = jnp.clip(sig0, 1e-7, 1.0 - 1e-7)
    cls_i = cls.astype(jnp.int32)
    onehot = c_iota == cls_i + 5
    chm = c_iota >= 5
    bce = -jnp.where(onehot, jnp.log(clsp), jnp.log(1.0 - clsp))
    cls_loss = jnp.sum(jnp.where(chm, bce, 0.0) * actf)
    back_total = jnp.sum(bl_ref[...]) + jnp.sum(bm_ref[...]) + jnp.sum(bh_ref[...])
    out_ref[0, 0] = (xy_loss + wh_loss + conf_fore + back_total + cls_loss) / _B


_INTERPRET = False


def _match_call(misc, targets):
    return pl.pallas_call(
        _match_body,
        grid=(1,),
        in_specs=[
            pl.BlockSpec(memory_space=pltpu.SMEM),
            pl.BlockSpec((_B, _NT, 5), lambda i: (0, 0, 0)),
        ],
        out_specs=[
            pl.BlockSpec((_B, _NT, 1), lambda i: (0, 0, 0)),
            pl.BlockSpec((_B, _NT, 1), lambda i: (0, 0, 0)),
        ],
        out_shape=[
            jax.ShapeDtypeStruct((_B, _NT, 1), jnp.int32),
            jax.ShapeDtypeStruct((_B, _NT, 1), jnp.float32),
        ],
        interpret=_INTERPRET,
    )(misc, targets)


def _dense_call(level, misc, data, targets, nt, valid):
    W = _GRIDW[level]
    H = W
    HW = W * H
    A0 = tuple((float(_ANCH[3 * level + a, 0]), float(_ANCH[3 * level + a, 1]))
               for a in range(3))
    body = functools.partial(_dense_body, W=W, H=H, OFF=_OFFS[level], A0=A0)
    return pl.pallas_call(
        body,
        grid=(_B,),
        in_specs=[
            pl.BlockSpec(memory_space=pltpu.SMEM),
            pl.BlockSpec((1, 3, 85, HW), lambda b: (b, 0, 0, 0)),
            pl.BlockSpec((1, _NT, 5), lambda b: (b, 0, 0)),
            pl.BlockSpec((1, _NT, 1), lambda b: (b, 0, 0)),
            pl.BlockSpec((1, _NT, 1), lambda b: (b, 0, 0)),
        ],
        out_specs=[
            pl.BlockSpec((1, 1, 1), lambda b: (b, 0, 0),
                         memory_space=pltpu.SMEM),
            pl.BlockSpec((1, _NT, 85), lambda b: (b, 0, 0)),
        ],
        out_shape=[
            jax.ShapeDtypeStruct((_B, 1, 1), jnp.float32),
            jax.ShapeDtypeStruct((_B, _NT, 85), jnp.float32),
        ],
        interpret=_INTERPRET,
    )(misc, data, targets, nt, valid)


def _loss_call(misc, targets, nt, valid, cl, cm, ch, bl, bm, bh):
    full = lambda shp: pl.BlockSpec(shp, lambda i: (0,) * len(shp))
    return pl.pallas_call(
        _loss_body,
        grid=(1,),
        in_specs=[
            pl.BlockSpec(memory_space=pltpu.SMEM),
            full((_B, _NT, 5)),
            full((_B, _NT, 1)),
            full((_B, _NT, 1)),
            full((_B, _NT, 85)),
            full((_B, _NT, 85)),
            full((_B, _NT, 85)),
            full((_B, 1, 1)),
            full((_B, 1, 1)),
            full((_B, 1, 1)),
        ],
        out_specs=pl.BlockSpec((1, 1), lambda i: (0, 0),
                               memory_space=pltpu.SMEM),
        out_shape=jax.ShapeDtypeStruct((1, 1), jnp.float32),
        interpret=_INTERPRET,
    )(misc, targets, nt, valid, cl, cm, ch, bl, bm, bh)


def kernel(l_data, m_data, h_data, targets, input_wh):
    iw_i = jnp.asarray(input_wh)
    iw_f = iw_i.astype(jnp.float32)
    zero = jnp.zeros((), jnp.float32)
    misc0 = jnp.stack([iw_f, zero])
    nt, valid = _match_call(misc0, targets)
    datas = (l_data, m_data, h_data)
    backs = []
    comps = []
    for level in range(3):
        W = _GRIDW[level]
        d = datas[level].reshape(_B, 3, 85, W * W)
        stride = (iw_i // W).astype(jnp.float32)
        misc = jnp.stack([iw_f, stride])
        b, c = _dense_call(level, misc, d, targets, nt, valid)
        backs.append(b)
        comps.append(c)
    out = _loss_call(misc0, targets, nt, valid, comps[0], comps[1], comps[2],
                     backs[0], backs[1], backs[2])
    return out[0, 0]


# speedup vs baseline: 11.5531x; 1.1025x over previous
import jax
import jax.numpy as jnp
import numpy as np
from jax.experimental import pallas as pl
from jax.experimental.pallas import tpu as pltpu

_ANCH = np.array(
    [[10, 13], [16, 30], [33, 23], [30, 61], [62, 45], [59, 119],
     [116, 90], [156, 198], [373, 326]], dtype=np.float32)
_GRIDW = (52, 26, 13)
_OFFS = (0, 8112, 10140)
_B = 8
_NT = 20


def _sel9(idx, vals):
    out = jnp.full(idx.shape, vals[8], dtype=jnp.float32)
    for k in range(7, -1, -1):
        out = jnp.where(idx == k, jnp.float32(vals[k]), out)
    return out


def _fused_body(misc_ref, l_ref, m_ref, h_ref, tgt_ref, out_ref):
    b = pl.program_id(0)
    iw = misc_ref[0]
    tgt = tgt_ref[0]
    x1 = tgt[:, 0:1]
    y1 = tgt[:, 1:2]
    x2 = tgt[:, 2:3]
    y2 = tgt[:, 3:4]
    cls = tgt[:, 4:5]
    w_n = x2 - x1
    h_n = y2 - y1
    vld = (w_n > 0) & (h_n > 0)
    cxn = (x1 + x2) * 0.5
    cyn = (y1 + y2) * 0.5
    w_px = w_n * iw
    h_px = h_n * iw

    ai = jax.lax.broadcasted_iota(jnp.int32, (_NT, 9), 1)
    aw9 = _sel9(ai, _ANCH[:, 0])
    ah9 = _sel9(ai, _ANCH[:, 1])
    ainter = jnp.minimum(w_px, aw9) * jnp.minimum(h_px, ah9)
    aiou = ainter / (w_px * h_px + aw9 * ah9 - ainter + 1e-9)
    mx = jnp.max(aiou, axis=1, keepdims=True)
    astar = jnp.clip(
        jnp.min(jnp.where(aiou == mx, ai, 99), axis=1, keepdims=True), 0, 8)
    s = astar // 3
    aloc = astar % 3
    gw = jnp.where(s == 0, _GRIDW[0], jnp.where(s == 1, _GRIDW[1], _GRIDW[2]))
    off = jnp.where(s == 0, _OFFS[0], jnp.where(s == 1, _OFFS[1], _OFFS[2]))
    gwf = gw.astype(jnp.float32)
    gi = jnp.clip((cxn * gwf).astype(jnp.int32), 0, gw - 1)
    gj = jnp.clip((cyn * gwf).astype(jnp.int32), 0, gw - 1)
    n = off + (gj * gw + gi) * 3 + aloc

    gx1 = x1 * iw
    gy1 = y1 * iw
    gx2 = x2 * iw
    gy2 = y2 * iw
    area_g = (gx2 - gx1) * (gy2 - gy1)

    back_sum = jnp.float32(0.0)
    comp = jnp.zeros((_NT, 85), jnp.float32)
    for level, ref in ((0, l_ref), (1, m_ref), (2, h_ref)):
        W = _GRIDW[level]
        HW = W * W
        OFF = _OFFS[level]
        stride = misc_ref[1 + level]
        pos = jax.lax.broadcasted_iota(jnp.int32, (1, HW), 1)
        gxf = (pos % W).astype(jnp.float32)
        gyf = (pos // W).astype(jnp.float32)
        for a in range(3):
            txs = jax.nn.sigmoid(ref[0, a, 0:1, :])
            tys = jax.nn.sigmoid(ref[0, a, 1:2, :])
            tw = ref[0, a, 2:3, :]
            th = ref[0, a, 3:4, :]
            conf_logit = ref[0, a, 4:5, :]
            cx = (txs + gxf) * stride
            cy = (tys + gyf) * stride
            aw = float(_ANCH[3 * level + a, 0])
            ah = float(_ANCH[3 * level + a, 1])
            bw = aw * jnp.exp(jnp.clip(tw, -10.0, 10.0))
            bh = ah * jnp.exp(jnp.clip(th, -10.0, 10.0))
            bx1 = cx - bw * 0.5
            by1 = cy - bh * 0.5
            bx2 = cx + bw * 0.5
            by2 = cy + bh * 0.5
            ix1 = jnp.maximum(bx1, gx1)
            iy1 = jnp.maximum(by1, gy1)
            ix2 = jnp.minimum(bx2, gx2)
            iy2 = jnp.minimum(by2, gy2)
            inter = (jnp.maximum(ix2 - ix1, 0.0)
                     * jnp.maximum(iy2 - iy1, 0.0))
            area_b = (bx2 - bx1) * (by2 - by1)
            iou = inter / (area_b + area_g - inter + 1e-9)
            max_iou = jnp.max(iou, axis=0, keepdims=True)
            back0 = max_iou <= 0.5
            n_glob = OFF + pos * 3 + a
            eq = n == n_glob
            fore = jnp.any(eq & vld, axis=0, keepdims=True)
            conf = jnp.clip(jax.nn.sigmoid(conf_logit), 1e-7, 1.0 - 1e-7)
            term = jnp.where(back0 & jnp.logical_not(fore),
                             -jnp.log(1.0 - conf), 0.0)
            back_sum = back_sum + jnp.sum(term)
            comp = comp + jax.lax.dot_general(
                eq.astype(jnp.float32), ref[0, a],
                (((1,), (1,)), ((), ())), preferred_element_type=jnp.float32)

    winner = jnp.full((_NT, 1), -1, jnp.int32)
    for tp in range(_NT):
        winner = jnp.where(vld[tp:tp + 1, :] & (n == n[tp:tp + 1, :]),
                           tp, winner)
    t_iota = jax.lax.broadcasted_iota(jnp.int32, (_NT, 1), 0)
    actf = (vld & (winner == t_iota)).astype(jnp.float32)

    awm = _sel9(astar, _ANCH[:, 0])
    ahm = _sel9(astar, _ANCH[:, 1])
    tx = cxn * gwf - gi.astype(jnp.float32)
    ty = cyn * gwf - gj.astype(jnp.float32)
    twt = jnp.log(jnp.maximum(w_px, 1.0) / awm)
    tht = jnp.log(jnp.maximum(h_px, 1.0) / ahm)
    scale = 2.0 - w_n * h_n

    sig0 = jax.nn.sigmoid(comp)
    px = sig0[:, 0:1]
    py = sig0[:, 1:2]
    pw = comp[:, 2:3]
    ph = comp[:, 3:4]
    pc = sig0[:, 4:5]
    sf = scale * actf
    xy_loss = jnp.sum(sf * ((px - tx) ** 2 + (py - ty) ** 2)) * 0.5
    wh_loss = jnp.sum(sf * ((pw - twt) ** 2 + (ph - tht) ** 2)) * 0.5
    pcc = jnp.clip(pc, 1e-7, 1.0 - 1e-7)
    conf_fore = jnp.sum(actf * (-jnp.log(pcc)))
    c_iota = jax.lax.broadcasted_iota(jnp.int32, (_NT, 85), 1)
    clsp = jnp.clip(sig0, 1e-7, 1.0 - 1e-7)
    onehot = c_iota == cls.astype(jnp.int32) + 5
    chm = c_iota >= 5
    bce = -jnp.where(onehot, jnp.log(clsp), jnp.log(1.0 - clsp))
    cls_loss = jnp.sum(jnp.where(chm, bce, 0.0) * actf)

    partial = xy_loss + wh_loss + conf_fore + cls_loss + back_sum
    prev = jnp.where(b == 0, 0.0, out_ref[0, 0, 0])
    tot = prev + partial
    out_ref[0, 0, 0] = jnp.where(b == _B - 1, tot / _B, tot)


_INTERPRET = False


def kernel(l_data, m_data, h_data, targets, input_wh):
    iw_i = jnp.asarray(input_wh)
    iw_f = iw_i.astype(jnp.float32)
    strides = [(iw_i // w).astype(jnp.float32) for w in _GRIDW]
    misc = jnp.stack([iw_f] + strides)
    dl = l_data.reshape(_B, 3, 85, _GRIDW[0] * _GRIDW[0])
    dm = m_data.reshape(_B, 3, 85, _GRIDW[1] * _GRIDW[1])
    dh = h_data.reshape(_B, 3, 85, _GRIDW[2] * _GRIDW[2])
    out = pl.pallas_call(
        _fused_body,
        grid=(_B,),
        in_specs=[
            pl.BlockSpec(memory_space=pltpu.SMEM),
            pl.BlockSpec((1, 3, 85, _GRIDW[0] ** 2), lambda b: (b, 0, 0, 0)),
            pl.BlockSpec((1, 3, 85, _GRIDW[1] ** 2), lambda b: (b, 0, 0, 0)),
            pl.BlockSpec((1, 3, 85, _GRIDW[2] ** 2), lambda b: (b, 0, 0, 0)),
            pl.BlockSpec((1, _NT, 5), lambda b: (b, 0, 0)),
        ],
        out_specs=pl.BlockSpec((1, 1, 1), lambda b: (0, 0, 0),
                               memory_space=pltpu.SMEM),
        out_shape=jax.ShapeDtypeStruct((1, 1, 1), jnp.float32),
        interpret=_INTERPRET,
    )(misc, dl, dm, dh, targets)
    return out[0, 0, 0]
